# baseline (speedup 1.0000x reference)
import functools

import jax
import jax.numpy as jnp
from jax.experimental import pallas as pl
from jax.experimental.pallas import tpu as pltpu
from jax.experimental.pallas import tpu_sc as plsc

TOPK = 8
TM = 256
TB = 256
GW = 128
SC_CHUNK = 256


def _router_body(x_ref, gw_ref, tw_ref, ti_ref, *, n_experts):
    x = x_ref[...]
    logits = jax.lax.dot_general(
        x, gw_ref[...], (((1,), (1,)), ((), ())),
        preferred_element_type=jnp.float32)
    m = jnp.max(logits, axis=1, keepdims=True)
    p = jnp.exp(logits - m)
    p = p / jnp.sum(p, axis=1, keepdims=True)

    iota = jax.lax.broadcasted_iota(jnp.int32, p.shape, 1)
    vals, ids = [], []
    for _ in range(TOPK):
        mk = jnp.max(p, axis=1, keepdims=True)
        idx = jnp.min(jnp.where(p == mk, iota, n_experts),
                      axis=1, keepdims=True)
        vals.append(mk)
        ids.append(idx)
        p = jnp.where(iota == idx, -1.0, p)
    tw = jnp.concatenate(vals, axis=1)
    tw_ref[...] = tw / jnp.sum(tw, axis=1, keepdims=True)
    ti_ref[...] = jnp.concatenate(ids, axis=1)


def _router(x, gate_weight):
    M, H = x.shape
    E = gate_weight.shape[0]
    return pl.pallas_call(
        functools.partial(_router_body, n_experts=E),
        grid=(M // TB,),
        in_specs=[
            pl.BlockSpec((TB, H), lambda i: (i, 0)),
            pl.BlockSpec((E, H), lambda i: (0, 0)),
        ],
        out_specs=[
            pl.BlockSpec((TB, TOPK), lambda i: (i, 0)),
            pl.BlockSpec((TB, TOPK), lambda i: (i, 0)),
        ],
        out_shape=[
            jax.ShapeDtypeStruct((M, TOPK), jnp.float32),
            jax.ShapeDtypeStruct((M, TOPK), jnp.int32),
        ],
    )(x, gate_weight)


def _sc_gather_rows(data, idx):
    P = idx.shape[0]
    W = data.shape[1]
    mesh = plsc.VectorSubcoreMesh(core_axis_name="c", subcore_axis_name="s")
    idx2 = idx.reshape(1, P)

    @pl.kernel(out_type=jax.ShapeDtypeStruct((P, W), data.dtype), mesh=mesh)
    def k(x_hbm, i_hbm, o_hbm):
        def body(i_vmem, o_vmem):
            pltpu.sync_copy(x_hbm.at[i_vmem.at[0]], o_vmem)

        pltpu.emit_pipeline(
            body,
            grid=(P // GW,),
            in_specs=[pl.BlockSpec((1, GW), index_map=lambda i: (0, i))],
            out_specs=[pl.BlockSpec((GW, W), index_map=lambda i: (i, 0))],
            core_axis_name=("c", "s"),
            dimension_semantics=(pltpu.PARALLEL,),
        )(i_hbm, o_hbm)

    return k(data, idx2)


def _sc_gather(data, idx):
    N, H = data.shape
    ns = H // SC_CHUNK
    data2 = data.reshape(N * ns, SC_CHUNK)
    idx_sub = (idx[:, None] * ns
               + jnp.arange(ns, dtype=jnp.int32)[None, :]).reshape(-1)
    out2 = _sc_gather_rows(data2, idx_sub)
    return out2.reshape(idx.shape[0], H)


def _ffn_body(tile_ref, exp_ref, start_ref, end_ref,
              x_ref, w13_ref, w2_ref, wrow_ref, y_ref, *, inter):
    g = pl.program_id(0)
    first = jnp.logical_or(g == 0, tile_ref[g] != tile_ref[jnp.maximum(g - 1, 0)])

    xb = x_ref[...].astype(jnp.bfloat16)
    w1 = w13_ref[0, :inter, :].astype(jnp.bfloat16)
    w3 = w13_ref[0, inter:, :].astype(jnp.bfloat16)
    w2 = w2_ref[0].astype(jnp.bfloat16)

    dn = (((1,), (1,)), ((), ()))
    a = jax.lax.dot_general(xb, w1, dn, preferred_element_type=jnp.float32)
    b = jax.lax.dot_general(xb, w3, dn, preferred_element_type=jnp.float32)
    h = (a * jax.nn.sigmoid(a) * b).astype(jnp.bfloat16)
    y = jax.lax.dot_general(h, w2, dn, preferred_element_type=jnp.float32)
    y = y * wrow_ref[...]

    iot = jax.lax.broadcasted_iota(jnp.int32, (TM, 1), 0)
    mask = jnp.logical_and(iot >= start_ref[g], iot < end_ref[g])
    yw = y

    @pl.when(first)
    def _():
        y_ref[...] = jnp.where(mask, yw, jnp.zeros_like(yw))

    @pl.when(jnp.logical_not(first))
    def _():
        y_ref[...] = jnp.where(mask, yw, y_ref[...])


def _grouped_ffn(x_sorted, w13, w2, wrow, tile_g, exp_g, start_g, end_g):
    P, H = x_sorted.shape
    E, two_i, _ = w13.shape
    inter = two_i // 2
    G = tile_g.shape[0]

    grid_spec = pltpu.PrefetchScalarGridSpec(
        num_scalar_prefetch=4,
        grid=(G,),
        in_specs=[
            pl.BlockSpec((TM, H), lambda g, t, e, s, en: (t[g], 0)),
            pl.BlockSpec((1, two_i, H), lambda g, t, e, s, en: (e[g], 0, 0)),
            pl.BlockSpec((1, H, inter), lambda g, t, e, s, en: (e[g], 0, 0)),
            pl.BlockSpec((TM, 1), lambda g, t, e, s, en: (t[g], 0)),
        ],
        out_specs=pl.BlockSpec((TM, H), lambda g, t, e, s, en: (t[g], 0)),
    )
    return pl.pallas_call(
        functools.partial(_ffn_body, inter=inter),
        grid_spec=grid_spec,
        out_shape=jax.ShapeDtypeStruct((P, H), jnp.float32),
        compiler_params=pltpu.CompilerParams(
            dimension_semantics=("arbitrary",)),
    )(tile_g, exp_g, start_g, end_g, x_sorted, w13, w2, wrow)


def _combine_body(y_ref, o_ref):
    o_ref[...] = jnp.sum(y_ref[...], axis=1)


def _combine_sum(y_flat3):
    M, K, H = y_flat3.shape
    return pl.pallas_call(
        _combine_body,
        grid=(M // TB,),
        in_specs=[pl.BlockSpec((TB, K, H), lambda i: (i, 0, 0))],
        out_specs=pl.BlockSpec((TB, H), lambda i: (i, 0)),
        out_shape=jax.ShapeDtypeStruct((M, H), jnp.float32),
    )(y_flat3)


def kernel(hidden_states, gate_weight, w13_stacked, w2_stacked):
    orig_shape = hidden_states.shape
    H = orig_shape[-1]
    x = hidden_states.reshape(-1, H)
    M = x.shape[0]
    E = gate_weight.shape[0]
    K = TOPK
    P = M * K
    T = P // TM
    G = T + E - 1

    topk_w, topk_ids = _router(x, gate_weight)

    flat_ids = topk_ids.reshape(-1)
    order = jnp.argsort(flat_ids).astype(jnp.int32)
    token_sorted = order // K
    pos = jnp.zeros((P,), jnp.int32).at[order].set(
        jnp.arange(P, dtype=jnp.int32))
    counts = jnp.bincount(flat_ids, length=E)
    off = jnp.concatenate([jnp.zeros((1,), counts.dtype),
                           jnp.cumsum(counts)]).astype(jnp.int32)

    t_lo = off[:-1] // TM
    t_hi = (off[1:] - 1) // TM
    tt = jnp.arange(T, dtype=jnp.int32)[:, None]
    visits = ((tt >= t_lo[None, :]) & (tt <= t_hi[None, :])
              & (counts[None, :] > 0))
    flat_v = visits.reshape(-1)
    ordv = jnp.argsort(jnp.logical_not(flat_v), stable=True)[:G]
    nvalid = jnp.sum(flat_v.astype(jnp.int32))
    valid = jnp.arange(G, dtype=jnp.int32) < nvalid
    tile_g = jnp.where(valid, ordv // E, T - 1).astype(jnp.int32)
    exp_g = jnp.where(valid, ordv % E, E - 1).astype(jnp.int32)
    start_g = jnp.where(
        valid, jnp.clip(off[exp_g] - tile_g * TM, 0, TM), 0).astype(jnp.int32)
    end_g = jnp.where(
        valid, jnp.clip(off[exp_g + 1] - tile_g * TM, 0, TM), 0).astype(jnp.int32)

    wrow = topk_w.reshape(-1)[order][:, None]

    x_sorted = _sc_gather(x, token_sorted)
    y_sorted = _grouped_ffn(x_sorted, w13_stacked, w2_stacked, wrow,
                            tile_g, exp_g, start_g, end_g)
    y_flat = _sc_gather(y_sorted, pos)
    out = _combine_sum(y_flat.reshape(M, K, H))
    return out.reshape(orig_shape).astype(hidden_states.dtype)


# speedup vs baseline: 1.1002x; 1.1002x over previous
import functools

import jax
import jax.numpy as jnp
from jax.experimental import pallas as pl
from jax.experimental.pallas import tpu as pltpu
from jax.experimental.pallas import tpu_sc as plsc

TOPK = 8
TM = 256
TB = 256
GW = 128
SC_CHUNK = 256


def _pack_bf16(v):
    c2 = v.shape[1] // 2
    u = jax.lax.bitcast_convert_type(v, jnp.uint32)
    u = u + jnp.uint32(0x7FFF) + ((u >> 16) & jnp.uint32(1))
    b = u >> 16
    packed = b[:, :c2] | (b[:, c2:] << 16)
    return jax.lax.bitcast_convert_type(packed, jnp.int32)


def _unpack_bf16(w):
    u = jax.lax.bitcast_convert_type(w, jnp.uint32)
    lo = jax.lax.bitcast_convert_type(u << 16, jnp.float32)
    hi = jax.lax.bitcast_convert_type(u & jnp.uint32(0xFFFF0000), jnp.float32)
    return jnp.concatenate([lo, hi], axis=1)


def _router_body(x_ref, gw_ref, tw_ref, ti_ref, rk_ref, cnt_ref, xp_ref, *, n_experts):
    x = x_ref[...]
    logits = jax.lax.dot_general(
        x, gw_ref[...], (((1,), (1,)), ((), ())),
        preferred_element_type=jnp.float32)
    m = jnp.max(logits, axis=1, keepdims=True)
    p = jnp.exp(logits - m)
    p = p / jnp.sum(p, axis=1, keepdims=True)

    iota = jax.lax.broadcasted_iota(jnp.int32, p.shape, 1)
    vals, ids = [], []
    for _ in range(TOPK):
        mk = jnp.max(p, axis=1, keepdims=True)
        idx = jnp.min(jnp.where(p == mk, iota, n_experts),
                      axis=1, keepdims=True)
        vals.append(mk)
        ids.append(idx)
        p = jnp.where(iota == idx, -1.0, p)
    tw = jnp.concatenate(vals, axis=1)
    tw_ref[...] = tw / jnp.sum(tw, axis=1, keepdims=True)
    ti_ref[...] = jnp.concatenate(ids, axis=1)

    onehots = [(iota == ids[k]).astype(jnp.float32) for k in range(TOPK)]
    rowcnt = sum(onehots)
    ri = jax.lax.broadcasted_iota(jnp.int32, (TB, TB), 0)
    ci = jax.lax.broadcasted_iota(jnp.int32, (TB, TB), 1)
    lt = (ri > ci).astype(jnp.float32)
    cum_rows = jax.lax.dot_general(
        lt, rowcnt, (((1,), (0,)), ((), ())),
        preferred_element_type=jnp.float32)
    ranks = []
    for k in range(TOPK):
        base = jnp.sum(onehots[k] * cum_rows, axis=1, keepdims=True)
        within = jnp.zeros_like(ids[k])
        for kp in range(k):
            within = within + (ids[kp] == ids[k]).astype(jnp.int32)
        ranks.append(base.astype(jnp.int32) + within)
    rk_ref[...] = jnp.concatenate(ranks, axis=1)
    cnt_ref[...] = jnp.sum(rowcnt, axis=0,
                           keepdims=True).astype(jnp.int32)[None]
    xp_ref[...] = _pack_bf16(x)


def _router(x, gate_weight):
    M, H = x.shape
    E = gate_weight.shape[0]
    nb = M // TB
    return pl.pallas_call(
        functools.partial(_router_body, n_experts=E),
        grid=(nb,),
        in_specs=[
            pl.BlockSpec((TB, H), lambda i: (i, 0)),
            pl.BlockSpec((E, H), lambda i: (0, 0)),
        ],
        out_specs=[
            pl.BlockSpec((TB, TOPK), lambda i: (i, 0)),
            pl.BlockSpec((TB, TOPK), lambda i: (i, 0)),
            pl.BlockSpec((TB, TOPK), lambda i: (i, 0)),
            pl.BlockSpec((1, 1, E), lambda i: (i, 0, 0)),
            pl.BlockSpec((TB, H // 2), lambda i: (i, 0)),
        ],
        out_shape=[
            jax.ShapeDtypeStruct((M, TOPK), jnp.float32),
            jax.ShapeDtypeStruct((M, TOPK), jnp.int32),
            jax.ShapeDtypeStruct((M, TOPK), jnp.int32),
            jax.ShapeDtypeStruct((nb, 1, E), jnp.int32),
            jax.ShapeDtypeStruct((M, H // 2), jnp.int32),
        ],
        compiler_params=pltpu.CompilerParams(
            dimension_semantics=("arbitrary",)),
    )(x, gate_weight)


def _sc_gather_rows(data, idx):
    P = idx.shape[0]
    W = data.shape[1]
    mesh = plsc.VectorSubcoreMesh(core_axis_name="c", subcore_axis_name="s")
    idx2 = idx.reshape(1, P)

    @pl.kernel(out_type=jax.ShapeDtypeStruct((P, W), data.dtype), mesh=mesh)
    def k(x_hbm, i_hbm, o_hbm):
        def body(i_vmem, o_vmem):
            pltpu.sync_copy(x_hbm.at[i_vmem.at[0]], o_vmem)

        pltpu.emit_pipeline(
            body,
            grid=(P // GW,),
            in_specs=[pl.BlockSpec((1, GW), index_map=lambda i: (0, i))],
            out_specs=[pl.BlockSpec((GW, W), index_map=lambda i: (i, 0))],
            core_axis_name=("c", "s"),
            dimension_semantics=(pltpu.PARALLEL,),
        )(i_hbm, o_hbm)

    return k(data, idx2)


def _sc_gather(data, idx):
    N, H = data.shape
    ns = H // SC_CHUNK
    data2 = data.reshape(N * ns, SC_CHUNK)
    idx_sub = (idx[:, None] * ns
               + jnp.arange(ns, dtype=jnp.int32)[None, :]).reshape(-1)
    out2 = _sc_gather_rows(data2, idx_sub)
    return out2.reshape(idx.shape[0], H)


def _ffn_body(tile_ref, exp_ref, start_ref, end_ref,
              x_ref, w13_ref, w2_ref, wrow_ref, y_ref, *, inter):
    g = pl.program_id(0)
    first = jnp.logical_or(g == 0, tile_ref[g] != tile_ref[jnp.maximum(g - 1, 0)])

    xb = _unpack_bf16(x_ref[...]).astype(jnp.bfloat16)
    w1 = w13_ref[0, :inter, :].astype(jnp.bfloat16)
    w3 = w13_ref[0, inter:, :].astype(jnp.bfloat16)
    w2 = w2_ref[0].astype(jnp.bfloat16)

    dn = (((1,), (1,)), ((), ()))
    a = jax.lax.dot_general(xb, w1, dn, preferred_element_type=jnp.float32)
    b = jax.lax.dot_general(xb, w3, dn, preferred_element_type=jnp.float32)
    h = (a * jax.nn.sigmoid(a) * b).astype(jnp.bfloat16)
    y = jax.lax.dot_general(h, w2, dn, preferred_element_type=jnp.float32)
    y = y * wrow_ref[...]

    iot = jax.lax.broadcasted_iota(jnp.int32, (TM, 1), 0)
    mask = jnp.logical_and(iot >= start_ref[g], iot < end_ref[g])
    yw = _pack_bf16(y)

    @pl.when(first)
    def _():
        y_ref[...] = jnp.where(mask, yw, jnp.zeros_like(yw))

    @pl.when(jnp.logical_not(first))
    def _():
        y_ref[...] = jnp.where(mask, yw, y_ref[...])


def _grouped_ffn(x_sorted, w13, w2, wrow, tile_g, exp_g, start_g, end_g):
    P, H2 = x_sorted.shape
    H = 2 * H2
    E, two_i, _ = w13.shape
    inter = two_i // 2
    G = tile_g.shape[0]

    grid_spec = pltpu.PrefetchScalarGridSpec(
        num_scalar_prefetch=4,
        grid=(G,),
        in_specs=[
            pl.BlockSpec((TM, H2), lambda g, t, e, s, en: (t[g], 0)),
            pl.BlockSpec((1, two_i, H), lambda g, t, e, s, en: (e[g], 0, 0)),
            pl.BlockSpec((1, H, inter), lambda g, t, e, s, en: (e[g], 0, 0)),
            pl.BlockSpec((TM, 1), lambda g, t, e, s, en: (t[g], 0)),
        ],
        out_specs=pl.BlockSpec((TM, H2), lambda g, t, e, s, en: (t[g], 0)),
    )
    return pl.pallas_call(
        functools.partial(_ffn_body, inter=inter),
        grid_spec=grid_spec,
        out_shape=jax.ShapeDtypeStruct((P, H2), jnp.int32),
        compiler_params=pltpu.CompilerParams(
            dimension_semantics=("arbitrary",)),
    )(tile_g, exp_g, start_g, end_g, x_sorted, w13, w2, wrow)


def _combine_body(y_ref, o_ref):
    u = jax.lax.bitcast_convert_type(y_ref[...], jnp.uint32)
    lo = jnp.sum(jax.lax.bitcast_convert_type(u << 16, jnp.float32), axis=1)
    hi = jnp.sum(jax.lax.bitcast_convert_type(
        u & jnp.uint32(0xFFFF0000), jnp.float32), axis=1)
    o_ref[...] = jnp.concatenate([lo, hi], axis=1)


def _combine_sum(y_flat3):
    M, K, H2 = y_flat3.shape
    H = 2 * H2
    return pl.pallas_call(
        _combine_body,
        grid=(M // TB,),
        in_specs=[pl.BlockSpec((TB, K, H2), lambda i: (i, 0, 0))],
        out_specs=pl.BlockSpec((TB, H), lambda i: (i, 0)),
        out_shape=jax.ShapeDtypeStruct((M, H), jnp.float32),
    )(y_flat3)


def kernel(hidden_states, gate_weight, w13_stacked, w2_stacked):
    orig_shape = hidden_states.shape
    H = orig_shape[-1]
    x = hidden_states.reshape(-1, H)
    M = x.shape[0]
    E = gate_weight.shape[0]
    K = TOPK
    P = M * K
    T = P // TM
    G = T + E - 1

    topk_w, topk_ids, rank, counts_be, x_packed = _router(x, gate_weight)

    counts_be = counts_be[:, 0, :]
    tot = jnp.sum(counts_be, axis=0)
    off = jnp.concatenate([jnp.zeros((1,), jnp.int32),
                           jnp.cumsum(tot)]).astype(jnp.int32)
    blk_excl = jnp.cumsum(counts_be, axis=0) - counts_be
    base_be = off[None, :E] + blk_excl
    blk_of_tok = (jnp.arange(M, dtype=jnp.int32) // TB)[:, None]
    pos = (base_be[blk_of_tok, topk_ids] + rank).reshape(-1)
    tokp = (jnp.arange(P, dtype=jnp.int32) // K)
    token_sorted = jnp.zeros((P,), jnp.int32).at[pos].set(tokp)
    wrow = jnp.zeros((P,), jnp.float32).at[pos].set(
        topk_w.reshape(-1))[:, None]

    t_lo = off[:E] // TM
    t_hi = (off[1:] - 1) // TM
    tiles_e = jnp.where(tot > 0, t_hi - t_lo + 1, 0)
    vb = jnp.concatenate([jnp.zeros((1,), jnp.int32),
                          jnp.cumsum(tiles_e)]).astype(jnp.int32)
    v = jnp.arange(G, dtype=jnp.int32)
    e_v = jnp.clip(jnp.searchsorted(vb, v, side="right") - 1, 0, E - 1)
    e_v = e_v.astype(jnp.int32)
    valid = v < vb[E]
    tile_g = jnp.where(valid,
                       jnp.clip(t_lo[e_v] + (v - vb[e_v]), 0, T - 1),
                       T - 1).astype(jnp.int32)
    exp_g = jnp.where(valid, e_v, E - 1).astype(jnp.int32)
    start_g = jnp.where(
        valid, jnp.clip(off[exp_g] - tile_g * TM, 0, TM), 0).astype(jnp.int32)
    end_g = jnp.where(
        valid, jnp.clip(off[exp_g + 1] - tile_g * TM, 0, TM), 0).astype(jnp.int32)

    x_sorted = _sc_gather(x_packed, token_sorted)
    y_sorted = _grouped_ffn(x_sorted, w13_stacked, w2_stacked, wrow,
                            tile_g, exp_g, start_g, end_g)
    y_flat = _sc_gather(y_sorted, pos)
    out = _combine_sum(y_flat.reshape(M, K, H // 2))
    return out.reshape(orig_shape).astype(hidden_states.dtype)
